# baseline (speedup 1.0000x reference)
import functools
import math

import jax
import jax.numpy as jnp
from jax import lax
from jax.experimental import pallas as pl
from jax.experimental.pallas import tpu as pltpu
from jax.experimental.pallas import tpu_sc as plsc

D_MODEL = 64
SCALE = math.sqrt(D_MODEL)
LANES = 16
BB = 128


@functools.lru_cache(maxsize=None)
def _build(S, NB):
    info = plsc.get_sparse_core_info()
    NC, NS = info.num_cores, info.num_subcores
    NW = NC * NS
    n_units = S * NB
    assert n_units % NW == 0
    U = n_units // NW
    assert U % 2 == 0 and U >= 6

    mesh = plsc.VectorSubcoreMesh(core_axis_name="c", subcore_axis_name="s")

    @functools.partial(
        pl.kernel,
        mesh=mesh,
        out_type=jax.ShapeDtypeStruct(
            (S, D_MODEL // 8, NB, 8, BB), jnp.float32
        ),
        compiler_params=pltpu.CompilerParams(
            use_tc_tiling_on_sc=False, needs_layout_passes=False
        ),
        scratch_types=[
            pltpu.VMEM((BB,), jnp.int32),
            pltpu.VMEM((BB,), jnp.int32),
            pltpu.VMEM((BB, D_MODEL // 2), jnp.float32),
            pltpu.VMEM((BB, D_MODEL // 2), jnp.float32),
            pltpu.VMEM((BB, D_MODEL // 2), jnp.float32),
            pltpu.VMEM((BB, D_MODEL // 2), jnp.float32),
            pltpu.VMEM((BB, D_MODEL + 1), jnp.float32),
            pltpu.VMEM((BB, D_MODEL + 1), jnp.float32),
            pltpu.VMEM((D_MODEL // 8, 8, BB), jnp.float32),
            pltpu.VMEM((D_MODEL // 8, 8, BB), jnp.float32),
            pltpu.SemaphoreType.DMA,
            pltpu.SemaphoreType.DMA,
            pltpu.SemaphoreType.DMA,
            pltpu.SemaphoreType.DMA,
            pltpu.SemaphoreType.DMA,
            pltpu.SemaphoreType.DMA,
            pltpu.SemaphoreType.DMA,
            pltpu.SemaphoreType.DMA,
        ],
    )
    def emb(xtl_hbm, tl_hbm, tr_hbm, out_hbm, ibuf0, ibuf1, gl0, gl1,
            gr0, gr1, gp0, gp1, o0, o1, isem0, isem1, glsem0, glsem1,
            grsem0, grsem1, osem0, osem1):
        ibuf = (ibuf0, ibuf1)
        gl = (gl0, gl1)
        gr = (gr0, gr1)
        gp = (gp0, gp1)
        ob = (o0, o1)
        isem = (isem0, isem1)
        glsem = (glsem0, glsem1)
        grsem = (grsem0, grsem1)
        osem = (osem0, osem1)

        wid = lax.axis_index("s") * NC + lax.axis_index("c")
        u0 = wid * U

        rowv = [
            lax.broadcasted_iota(jnp.int32, (LANES,), 0) + g * LANES
            for g in range(BB // LANES)
        ]

        def unit_su(u):
            ug = u0 + u
            return ug // NB, lax.rem(ug, NB)

        def issue_idx(u, b):
            s, bhi = unit_su(u)
            pltpu.async_copy(
                xtl_hbm.at[pl.ds(s * (NB * BB) + bhi * BB, BB)],
                ibuf[b], isem[b],
            )

        def wait_idx(u, b):
            s, bhi = unit_su(u)
            pltpu.make_async_copy(
                xtl_hbm.at[pl.ds(s * (NB * BB) + bhi * BB, BB)],
                ibuf[b], isem[b],
            ).wait()

        def issue_gather(b):
            pltpu.async_copy(tl_hbm.at[ibuf[b]], gl[b], glsem[b])
            pltpu.async_copy(tr_hbm.at[ibuf[b]], gr[b], grsem[b])

        def wait_gather(b):
            pltpu.make_async_copy(tl_hbm.at[ibuf[b]], gl[b], glsem[b]).wait()
            pltpu.make_async_copy(tr_hbm.at[ibuf[b]], gr[b], grsem[b]).wait()

        def issue_out(u, b):
            s, bhi = unit_su(u)
            pltpu.async_copy(ob[b], out_hbm.at[s, :, bhi], osem[b])

        def wait_out(u, b):
            s, bhi = unit_su(u)
            pltpu.make_async_copy(
                ob[b], out_hbm.at[s, :, bhi], osem[b]
            ).wait()

        def transpose(b):
            @plsc.parallel_loop(0, BB, unroll=16)
            def rp(r):
                for j in range(D_MODEL // (2 * LANES)):
                    sl = pl.ds(j * LANES, LANES)
                    gp[b][r, sl] = gl[b][r, sl]
                    gp[b][r, pl.ds(D_MODEL // 2 + j * LANES, LANES)] = (
                        gr[b][r, sl]
                    )

            @plsc.parallel_loop(0, D_MODEL, unroll=16)
            def tp(d):
                dhi = lax.shift_right_logical(d, 3)
                dlo = lax.bitwise_and(d, 7)
                cv = jnp.full((LANES,), d, jnp.int32)
                for g in range(BB // LANES):
                    v = plsc.load_gather(gp[b], [rowv[g], cv])
                    ob[b][dhi, dlo, pl.ds(g * LANES, LANES)] = v * SCALE

        issue_idx(0, 0)
        issue_idx(1, 1)
        wait_idx(0, 0)
        issue_gather(0)
        for u in range(2):
            b = u % 2
            wait_gather(b)
            wait_idx(u + 1, 1 - b)
            issue_gather(1 - b)
            issue_idx(u + 2, b)
            transpose(b)
            issue_out(u, b)

        def pair(i, carry):
            for b in range(2):
                u = 2 * i + b
                wait_gather(b)
                wait_idx(u + 1, 1 - b)
                issue_gather(1 - b)
                issue_idx(u + 2, b)
                wait_out(u - 2, b)
                transpose(b)
                issue_out(u, b)
            return carry

        lax.fori_loop(1, U // 2 - 1, pair, 0)

        u = U - 2
        wait_gather(0)
        wait_idx(u + 1, 1)
        issue_gather(1)
        wait_out(u - 2, 0)
        transpose(0)
        issue_out(u, 0)

        u = U - 1
        wait_gather(1)
        wait_out(u - 2, 1)
        transpose(1)
        issue_out(u, 1)

        wait_out(U - 2, 0)
        wait_out(U - 1, 1)

    return emb


def kernel(x, table):
    B0, S = x.shape
    NB = B0 // BB
    V, D = table.shape
    tl = lax.slice(table, (0, 0), (V, D // 2))
    tr = lax.slice(table, (0, D // 2), (V, D))
    xtl = jnp.transpose(x).reshape(-1).astype(jnp.int32)
    out5 = _build(S, NB)(xtl, tl, tr)
    out = jnp.transpose(out5, (2, 4, 0, 1, 3)).reshape(B0, S, D_MODEL)
    return out


# speedup vs baseline: 1.9103x; 1.9103x over previous
import functools
import math

import jax
import jax.numpy as jnp
from jax import lax
from jax.experimental import pallas as pl
from jax.experimental.pallas import tpu as pltpu
from jax.experimental.pallas import tpu_sc as plsc

D_MODEL = 64
SCALE = math.sqrt(D_MODEL)
LANES = 16
BB = 128


@functools.lru_cache(maxsize=None)
def _build(S, NB):
    info = plsc.get_sparse_core_info()
    NC, NS = info.num_cores, info.num_subcores
    NW = NC * NS
    n_units = S * NB
    assert n_units % NW == 0
    U = n_units // NW
    assert U % 2 == 0 and U >= 6

    mesh = plsc.VectorSubcoreMesh(core_axis_name="c", subcore_axis_name="s")

    @functools.partial(
        pl.kernel,
        mesh=mesh,
        out_type=jax.ShapeDtypeStruct(
            (S, D_MODEL // 8, NB, 8, BB), jnp.float32
        ),
        compiler_params=pltpu.CompilerParams(
            use_tc_tiling_on_sc=False, needs_layout_passes=False
        ),
        scratch_types=[
            pltpu.VMEM((BB,), jnp.int32),
            pltpu.VMEM((BB,), jnp.int32),
            pltpu.VMEM((BB, D_MODEL), jnp.float32),
            pltpu.VMEM((BB, D_MODEL), jnp.float32),
            pltpu.VMEM((BB, D_MODEL + 1), jnp.float32),
            pltpu.VMEM((BB, D_MODEL + 1), jnp.float32),
            pltpu.VMEM((D_MODEL // 8, 8, BB), jnp.float32),
            pltpu.VMEM((D_MODEL // 8, 8, BB), jnp.float32),
            pltpu.SemaphoreType.DMA,
            pltpu.SemaphoreType.DMA,
            pltpu.SemaphoreType.DMA,
            pltpu.SemaphoreType.DMA,
            pltpu.SemaphoreType.DMA,
            pltpu.SemaphoreType.DMA,
        ],
    )
    def emb(xtl_hbm, table_hbm, out_hbm, ibuf0, ibuf1, g0, g1, gp0, gp1,
            o0, o1, isem0, isem1, gsem0, gsem1, osem0, osem1):
        ibuf = (ibuf0, ibuf1)
        gb = (g0, g1)
        gp = (gp0, gp1)
        ob = (o0, o1)
        isem = (isem0, isem1)
        gsem = (gsem0, gsem1)
        osem = (osem0, osem1)

        wid = lax.axis_index("s") * NC + lax.axis_index("c")
        u0 = wid * U

        rowv = [
            lax.broadcasted_iota(jnp.int32, (LANES,), 0) + g * LANES
            for g in range(BB // LANES)
        ]

        def unit_su(u):
            ug = u0 + u
            return ug // NB, lax.rem(ug, NB)

        def issue_idx(u, b):
            s, bhi = unit_su(u)
            pltpu.async_copy(
                xtl_hbm.at[pl.ds(s * (NB * BB) + bhi * BB, BB)],
                ibuf[b], isem[b],
            )

        def wait_idx(u, b):
            s, bhi = unit_su(u)
            pltpu.make_async_copy(
                xtl_hbm.at[pl.ds(s * (NB * BB) + bhi * BB, BB)],
                ibuf[b], isem[b],
            ).wait()

        def issue_gather(b):
            pltpu.async_copy(table_hbm.at[ibuf[b]], gb[b], gsem[b])

        def wait_gather(b):
            pltpu.make_async_copy(
                table_hbm.at[ibuf[b]], gb[b], gsem[b]
            ).wait()

        def issue_out(u, b):
            s, bhi = unit_su(u)
            pltpu.async_copy(ob[b], out_hbm.at[s, :, bhi], osem[b])

        def wait_out(u, b):
            s, bhi = unit_su(u)
            pltpu.make_async_copy(
                ob[b], out_hbm.at[s, :, bhi], osem[b]
            ).wait()

        def transpose(b):
            @plsc.parallel_loop(0, BB, unroll=16)
            def rp(r):
                for j in range(D_MODEL // LANES):
                    sl = pl.ds(j * LANES, LANES)
                    gp[b][r, sl] = gb[b][r, sl]

            @plsc.parallel_loop(0, D_MODEL, unroll=16)
            def tp(d):
                dhi = lax.shift_right_logical(d, 3)
                dlo = lax.bitwise_and(d, 7)
                cv = jnp.full((LANES,), d, jnp.int32)
                for g in range(BB // LANES):
                    v = plsc.load_gather(gp[b], [rowv[g], cv])
                    ob[b][dhi, dlo, pl.ds(g * LANES, LANES)] = v * SCALE

        issue_idx(0, 0)
        issue_idx(1, 1)
        wait_idx(0, 0)
        issue_gather(0)
        for u in range(2):
            b = u % 2
            wait_gather(b)
            wait_idx(u + 1, 1 - b)
            issue_gather(1 - b)
            issue_idx(u + 2, b)
            transpose(b)
            issue_out(u, b)

        def pair(i, carry):
            for b in range(2):
                u = 2 * i + b
                wait_gather(b)
                wait_idx(u + 1, 1 - b)
                issue_gather(1 - b)
                issue_idx(u + 2, b)
                wait_out(u - 2, b)
                transpose(b)
                issue_out(u, b)
            return carry

        lax.fori_loop(1, U // 2 - 1, pair, 0)

        u = U - 2
        wait_gather(0)
        wait_idx(u + 1, 1)
        issue_gather(1)
        wait_out(u - 2, 0)
        transpose(0)
        issue_out(u, 0)

        u = U - 1
        wait_gather(1)
        wait_out(u - 2, 1)
        transpose(1)
        issue_out(u, 1)

        wait_out(U - 2, 0)
        wait_out(U - 1, 1)

    return emb


def kernel(x, table):
    B0, S = x.shape
    NB = B0 // BB
    xtl = jnp.transpose(x).reshape(-1).astype(jnp.int32)
    out5 = _build(S, NB)(xtl, table)
    out = jnp.transpose(out5, (2, 4, 0, 1, 3)).reshape(B0, S, D_MODEL)
    return out
